# Initial kernel scaffold; baseline (speedup 1.0000x reference)
import jax
import jax.numpy as jnp
from jax.experimental import pallas as pl


def kernel(x, target):
    raise NotImplementedError("write your pallas kernel here")


# speedup vs baseline: 1.7596x; 1.7596x over previous
import jax
import jax.numpy as jnp
from jax.experimental import pallas as pl
from jax.experimental.pallas import tpu as pltpu

_V = 100000
_N = 1024
_PAD = 0
_SMOOTH = 0.1
_CONF = 1.0 - _SMOOTH
_FILL = _SMOOTH / (_V - 2)

_VB = 2048
_NVB = (_V + _VB - 1) // _VB

import math
_C = _FILL * math.log(_FILL) * (_V - 2) + _CONF * math.log(_CONF)


def _loss_kernel(x_ref, tgt_ref, out_ref):
    k = pl.program_id(0)
    xb = x_ref[...]
    base = k * _VB
    col = jax.lax.broadcasted_iota(jnp.int32, (_N, _VB), 1) + base
    xb = jnp.where(col < _V, xb, 0.0)

    tgt = tgt_ref[...]
    valid = tgt != _PAD

    rowsum = jnp.sum(xb, axis=1, keepdims=True)
    tgt_v = jnp.sum(jnp.where(col == tgt, xb, 0.0), axis=1, keepdims=True)
    x0_v = jnp.sum(jnp.where(col == _PAD, xb, 0.0), axis=1, keepdims=True)

    partial = -_FILL * rowsum + _FILL * x0_v + (_FILL - _CONF) * tgt_v
    partial = jnp.where(valid, partial, 0.0)
    s = jnp.sum(partial)

    @pl.when(k == 0)
    def _init():
        nvalid = jnp.sum(valid.astype(jnp.float32))
        out_ref[0, 0] = nvalid * _C

    out_ref[0, 0] += s


def kernel(x, target):
    tgt2 = target.reshape(_N, 1)
    out = pl.pallas_call(
        _loss_kernel,
        grid=(_NVB,),
        in_specs=[
            pl.BlockSpec((_N, _VB), lambda k: (0, k)),
            pl.BlockSpec((_N, 1), lambda k: (0, 0)),
        ],
        out_specs=pl.BlockSpec((1, 1), lambda k: (0, 0),
                               memory_space=pltpu.SMEM),
        out_shape=jax.ShapeDtypeStruct((1, 1), jnp.float32),
    )(x, tgt2)
    return out[0, 0]
